# Initial kernel scaffold; baseline (speedup 1.0000x reference)
import jax
import jax.numpy as jnp
from jax.experimental import pallas as pl


def kernel(inputs, weight):
    raise NotImplementedError("write your pallas kernel here")


# speedup vs baseline: 1.7986x; 1.7986x over previous
import functools

import jax
import jax.numpy as jnp
from jax import lax
from jax.experimental import pallas as pl
from jax.experimental.pallas import tpu as pltpu
from jax.experimental.pallas import tpu_sc as plsc

HIDDEN = 64
NUM_CORES = 2
NUM_SUBCORES = 16
NW = NUM_CORES * NUM_SUBCORES
CH = 128
K = 4
G = CH * K

_mesh = plsc.VectorSubcoreMesh(core_axis_name="c", subcore_axis_name="s")


@functools.lru_cache(maxsize=None)
def _make_gather(n):
    assert n % (NW * G) == 0
    per_w = n // NW
    groups = per_w // G

    @functools.partial(
        pl.kernel,
        out_type=jax.ShapeDtypeStruct((n, HIDDEN), jnp.float32),
        mesh=_mesh,
        scratch_types=[
            pltpu.VMEM((K, CH), jnp.int32),
            pltpu.VMEM((G, HIDDEN), jnp.float32),
            pltpu.SemaphoreType.DMA,
        ],
        compiler_params=pltpu.CompilerParams(use_tc_tiling_on_sc=False),
    )
    def gather_kernel(table_hbm, idx_hbm, out_hbm, idx_v, rows_v, gsem):
        wid = lax.axis_index("s") * NUM_CORES + lax.axis_index("c")
        row0 = wid * (per_w // CH)
        base = wid * per_w

        @pl.loop(0, groups)
        def _group(g):
            pltpu.sync_copy(idx_hbm.at[pl.ds(row0 + g * K, K)], idx_v)
            copies = [
                pltpu.async_copy(
                    table_hbm.at[idx_v.at[j]],
                    rows_v.at[pl.ds(j * CH, CH)],
                    gsem,
                )
                for j in range(K)
            ]
            for c in copies:
                c.wait()
            pltpu.sync_copy(rows_v, out_hbm.at[pl.ds(base + g * G, G)])

    return gather_kernel


def kernel(inputs, weight):
    batch, hist = inputs.shape
    n = batch * hist
    pad = (-n) % (NW * G)
    idx = inputs.reshape(-1).astype(jnp.int32)
    if pad:
        idx = jnp.pad(idx, (0, pad))
    idx = idx.reshape((n + pad) // CH, CH)
    out = _make_gather(n + pad)(weight, idx)
    return out[:n].reshape(batch, hist, HIDDEN)


# speedup vs baseline: 1.8517x; 1.0295x over previous
import functools

import jax
import jax.numpy as jnp
from jax import lax
from jax.experimental import pallas as pl
from jax.experimental.pallas import tpu as pltpu
from jax.experimental.pallas import tpu_sc as plsc

HIDDEN = 64
NUM_CORES = 2
NUM_SUBCORES = 16
NW = NUM_CORES * NUM_SUBCORES
CH = 128
K = 4
G = CH * K

_mesh = plsc.VectorSubcoreMesh(core_axis_name="c", subcore_axis_name="s")


@functools.lru_cache(maxsize=None)
def _make_gather(n):
    assert n % (NW * G * 2) == 0
    per_w = n // NW
    groups = per_w // G

    @functools.partial(
        pl.kernel,
        out_type=jax.ShapeDtypeStruct((n, HIDDEN), jnp.float32),
        mesh=_mesh,
        scratch_types=[
            pltpu.VMEM((2, K, CH), jnp.int32),
            pltpu.VMEM((2, G, HIDDEN), jnp.float32),
            pltpu.SemaphoreType.DMA,
            pltpu.SemaphoreType.DMA,
            pltpu.SemaphoreType.DMA,
            pltpu.SemaphoreType.DMA,
        ],
        compiler_params=pltpu.CompilerParams(use_tc_tiling_on_sc=False),
    )
    def gather_kernel(table_hbm, idx_hbm, out_hbm, idx_v, rows_v,
                      gsem0, gsem1, osem0, osem1):
        wid = lax.axis_index("s") * NUM_CORES + lax.axis_index("c")
        row0 = wid * (per_w // CH)
        base = wid * per_w
        gsems = (gsem0, gsem1)
        osems = (osem0, osem1)

        def fire(b, g):
            pltpu.sync_copy(idx_hbm.at[pl.ds(row0 + g * K, K)], idx_v.at[b])
            for j in range(K):
                pltpu.async_copy(table_hbm.at[idx_v.at[b].at[j]],
                                 rows_v.at[b].at[pl.ds(j * CH, CH)], gsems[b])

        def drain_gathers(b):
            for j in range(K):
                pltpu.make_async_copy(table_hbm.at[idx_v.at[b].at[j]],
                                      rows_v.at[b].at[pl.ds(j * CH, CH)],
                                      gsems[b]).wait()

        def fire_out(b, g):
            pltpu.async_copy(rows_v.at[b], out_hbm.at[pl.ds(base + g * G, G)],
                             osems[b])

        def wait_out(b, g):
            pltpu.make_async_copy(rows_v.at[b],
                                  out_hbm.at[pl.ds(base + g * G, G)],
                                  osems[b]).wait()

        fire(0, 0)

        @pl.loop(0, groups, step=2)
        def _grp(g0):
            for b in (0, 1):
                gg = g0 + b
                nb = 1 - b
                gf = gg + 1

                @pl.when(gf < groups)
                def _():
                    @pl.when(gg >= 1)
                    def _():
                        wait_out(nb, gg - 1)
                    fire(nb, gf)

                drain_gathers(b)
                fire_out(b, gg)

        wait_out(0, groups - 2)
        wait_out(1, groups - 1)

    return gather_kernel


def kernel(inputs, weight):
    batch, hist = inputs.shape
    n = batch * hist
    pad = (-n) % (NW * G * 2)
    idx = inputs.reshape(-1).astype(jnp.int32)
    if pad:
        idx = jnp.pad(idx, (0, pad))
    idx = idx.reshape((n + pad) // CH, CH)
    out = _make_gather(n + pad)(weight, idx)
    return out[:n].reshape(batch, hist, HIDDEN)


# speedup vs baseline: 2.4592x; 1.3281x over previous
import functools

import jax
import jax.numpy as jnp
from jax import lax
from jax.experimental import pallas as pl
from jax.experimental.pallas import tpu as pltpu
from jax.experimental.pallas import tpu_sc as plsc

HIDDEN = 64
NUM_CORES = 2
NUM_SUBCORES = 16
NW = NUM_CORES * NUM_SUBCORES
NB = 8

_mesh = plsc.VectorSubcoreMesh(core_axis_name="c", subcore_axis_name="s")


@functools.lru_cache(maxsize=None)
def _make_gather(batch, hist):
    assert batch % (NW * NB * 2) == 0 and hist <= 128
    hp = ((hist + 7) // 8) * 8
    per_w = batch // NW
    groups = per_w // NB
    gr = NB * hp

    @functools.partial(
        pl.kernel,
        out_type=jax.ShapeDtypeStruct((batch * hp, 2 * HIDDEN), jnp.float32),
        mesh=_mesh,
        scratch_types=[
            pltpu.VMEM((2, NB, hist), jnp.int32),
            pltpu.VMEM((2, gr, HIDDEN), jnp.float32),
            pltpu.SemaphoreType.DMA,
            pltpu.SemaphoreType.DMA,
            pltpu.SemaphoreType.DMA,
            pltpu.SemaphoreType.DMA,
        ],
        compiler_params=pltpu.CompilerParams(use_tc_tiling_on_sc=False),
    )
    def gather_kernel(table_hbm, idx_hbm, out_hbm, idx_v, rows_v,
                      gsem0, gsem1, osem0, osem1):
        wid = lax.axis_index("s") * NUM_CORES + lax.axis_index("c")
        base_b = wid * per_w
        base_r = base_b * hp
        gsems = (gsem0, gsem1)
        osems = (osem0, osem1)

        def fire(b, g):
            pltpu.sync_copy(idx_hbm.at[pl.ds(base_b + g * NB, NB)], idx_v.at[b])
            for j in range(NB):
                pltpu.async_copy(table_hbm.at[idx_v.at[b].at[j]],
                                 rows_v.at[b].at[pl.ds(j * hp, hist)],
                                 gsems[b])

        def drain_gathers(b):
            for j in range(NB):
                pltpu.make_async_copy(table_hbm.at[idx_v.at[b].at[j]],
                                      rows_v.at[b].at[pl.ds(j * hp, hist)],
                                      gsems[b]).wait()

        def out_slice(g):
            return out_hbm.at[pl.ds(base_r + g * gr, gr), pl.ds(0, HIDDEN)]

        def fire_out(b, g):
            pltpu.async_copy(rows_v.at[b], out_slice(g), osems[b])

        def wait_out(b, g):
            pltpu.make_async_copy(rows_v.at[b], out_slice(g), osems[b]).wait()

        fire(0, 0)

        @pl.loop(0, groups, step=2)
        def _grp(g0):
            for b in (0, 1):
                gg = g0 + b
                nb = 1 - b
                gf = gg + 1

                @pl.when(gf < groups)
                def _():
                    @pl.when(gg >= 1)
                    def _():
                        wait_out(nb, gg - 1)
                    fire(nb, gf)

                drain_gathers(b)
                fire_out(b, gg)

        wait_out(0, groups - 2)
        wait_out(1, groups - 1)

    return gather_kernel


def kernel(inputs, weight):
    batch, hist = inputs.shape
    hp = ((hist + 7) // 8) * 8
    idx = inputs.astype(jnp.int32)
    out = _make_gather(batch, hist)(weight, idx)
    return out.reshape(batch, hp, 2 * HIDDEN)[:, :hist, :HIDDEN]


# speedup vs baseline: 2.4849x; 1.0104x over previous
import functools

import jax
import jax.numpy as jnp
from jax import lax
from jax.experimental import pallas as pl
from jax.experimental.pallas import tpu as pltpu
from jax.experimental.pallas import tpu_sc as plsc

HIDDEN = 64
NUM_CORES = 2
NUM_SUBCORES = 16
NW = NUM_CORES * NUM_SUBCORES
NB = 16

_mesh = plsc.VectorSubcoreMesh(core_axis_name="c", subcore_axis_name="s")


@functools.lru_cache(maxsize=None)
def _make_gather(batch, hist):
    assert batch % (NW * NB * 2) == 0 and hist <= 128
    hp = ((hist + 7) // 8) * 8
    per_w = batch // NW
    groups = per_w // NB
    gr = NB * hp

    @functools.partial(
        pl.kernel,
        out_type=jax.ShapeDtypeStruct((batch * hp, 2 * HIDDEN), jnp.float32),
        mesh=_mesh,
        scratch_types=[
            pltpu.VMEM((2, NB, hist), jnp.int32),
            pltpu.VMEM((2, gr, HIDDEN), jnp.float32),
            pltpu.SemaphoreType.DMA,
            pltpu.SemaphoreType.DMA,
            pltpu.SemaphoreType.DMA,
            pltpu.SemaphoreType.DMA,
        ],
        compiler_params=pltpu.CompilerParams(use_tc_tiling_on_sc=False),
    )
    def gather_kernel(table_hbm, idx_hbm, out_hbm, idx_v, rows_v,
                      gsem0, gsem1, osem0, osem1):
        wid = lax.axis_index("s") * NUM_CORES + lax.axis_index("c")
        base_b = wid * per_w
        base_r = base_b * hp
        gsems = (gsem0, gsem1)
        osems = (osem0, osem1)

        def fire(b, g):
            pltpu.sync_copy(idx_hbm.at[pl.ds(base_b + g * NB, NB)], idx_v.at[b])
            for j in range(NB):
                pltpu.async_copy(table_hbm.at[idx_v.at[b].at[j]],
                                 rows_v.at[b].at[pl.ds(j * hp, hist)],
                                 gsems[b])

        def drain_gathers(b):
            for j in range(NB):
                pltpu.make_async_copy(table_hbm.at[idx_v.at[b].at[j]],
                                      rows_v.at[b].at[pl.ds(j * hp, hist)],
                                      gsems[b]).wait()

        def out_slice(g):
            return out_hbm.at[pl.ds(base_r + g * gr, gr), pl.ds(0, HIDDEN)]

        def fire_out(b, g):
            pltpu.async_copy(rows_v.at[b], out_slice(g), osems[b])

        def wait_out(b, g):
            pltpu.make_async_copy(rows_v.at[b], out_slice(g), osems[b]).wait()

        fire(0, 0)

        @pl.loop(0, groups, step=2)
        def _grp(g0):
            for b in (0, 1):
                gg = g0 + b
                nb = 1 - b
                gf = gg + 1

                @pl.when(gf < groups)
                def _():
                    @pl.when(gg >= 1)
                    def _():
                        wait_out(nb, gg - 1)
                    fire(nb, gf)

                drain_gathers(b)
                fire_out(b, gg)

        wait_out(0, groups - 2)
        wait_out(1, groups - 1)

    return gather_kernel


def kernel(inputs, weight):
    batch, hist = inputs.shape
    hp = ((hist + 7) // 8) * 8
    idx = inputs.astype(jnp.int32)
    out = _make_gather(batch, hist)(weight, idx)
    return out.reshape(batch, hp, 2 * HIDDEN)[:, :hist, :HIDDEN]
